# baseline (speedup 1.0000x reference)
import functools

import jax
import jax.numpy as jnp
from jax import lax
from jax.experimental import pallas as pl
from jax.experimental.pallas import tpu as pltpu
from jax.experimental.pallas import tpu_sc as plsc

_B, _S = 1024, 200
_T = _B * _S
_DI = 64
_DM = 128
_V = 1000001

_H = 501760
_PACK_IB = 17920
_NPACK = _H // _PACK_IB

_NW = 32
_IDX_COLS = 128
_TH = _T // 2
_TOK_PER_W = _TH // _NW
_CHUNK_TOK = 640
_NCHUNK = _TOK_PER_W // _CHUNK_TOK
_GPC = _CHUNK_TOK // _IDX_COLS


def _pack_body(w1_ref, w2_ref, out_ref):
    xs = jnp.concatenate([w1_ref[...], w2_ref[...]], axis=0)
    eye = (lax.broadcasted_iota(jnp.int32, (_DM, _DM), 0)
           == lax.broadcasted_iota(jnp.int32, (_DM, _DM), 1)).astype(jnp.float32)
    out_ref[...] = lax.dot_general(
        xs, eye, (((0,), (0,)), ((), ())),
        preferred_element_type=jnp.float32)


@jax.jit
def _tc_pack(wt):
    return pl.pallas_call(
        _pack_body,
        grid=(_NPACK,),
        in_specs=[
            pl.BlockSpec((_DI, _PACK_IB), lambda i: (0, i)),
            pl.BlockSpec((_DI, _PACK_IB),
                         lambda i: (0, jnp.minimum(_NPACK + i, 2 * _NPACK - 1))),
        ],
        out_specs=pl.BlockSpec((_PACK_IB, _DM), lambda i: (i, 0)),
        out_shape=jax.ShapeDtypeStruct((_H, _DM), jnp.float32),
    )(wt, wt)


def _sc_gather_body(table_hbm, idx_hbm, out_hbm,
                    i0, i1, i2, i3, i4, rows_v, isem, sem):
    nc = 2
    wid = lax.axis_index("s") * nc + lax.axis_index("c")
    tok0 = wid * _TOK_PER_W
    idx_bufs = (i0, i1, i2, i3, i4)
    for g in range(_NCHUNK):
        base = tok0 + g * _CHUNK_TOK
        ic = []
        for j in range(_GPC):
            ic.append(
                pltpu.async_copy(
                    idx_hbm.at[pl.ds(base + j * _IDX_COLS, _IDX_COLS)],
                    idx_bufs[j], isem,
                )
            )
        for c in ic:
            c.wait()
        copies = []
        for j in range(_GPC):
            copies.append(
                pltpu.async_copy(
                    table_hbm.at[idx_bufs[j]],
                    rows_v.at[pl.ds(j * _IDX_COLS, _IDX_COLS)],
                    sem,
                )
            )
        for c in copies:
            c.wait()
        pltpu.sync_copy(rows_v, out_hbm.at[pl.ds(base, _CHUNK_TOK)])


@jax.jit
def _sc_gather(table, idx1d):
    mesh = plsc.VectorSubcoreMesh(core_axis_name="c", subcore_axis_name="s")
    f = functools.partial(
        pl.kernel,
        mesh=mesh,
        out_type=jax.ShapeDtypeStruct((_TH, _DM), jnp.float32),
        scratch_types=[
            pltpu.VMEM((_IDX_COLS,), jnp.int32),
            pltpu.VMEM((_IDX_COLS,), jnp.int32),
            pltpu.VMEM((_IDX_COLS,), jnp.int32),
            pltpu.VMEM((_IDX_COLS,), jnp.int32),
            pltpu.VMEM((_IDX_COLS,), jnp.int32),
            pltpu.VMEM((_CHUNK_TOK, _DM), jnp.float32),
            pltpu.SemaphoreType.DMA,
            pltpu.SemaphoreType.DMA,
        ],
    )(_sc_gather_body)
    return f(table, idx1d)


_BB = 16
_TROWS = _BB * _S
_BH = _B // 2
_NTILE = _BH // _BB


def _tc_body(g_ref, part_ref, corr_ref, elap_ref, half_ref, msrc_ref, pos_ref,
             wagg_ref, bagg_ref, out_ref):
    A = wagg_ref[...]
    g = g_ref[...]

    m = jnp.dot(msrc_ref[...], A[_DI:112, :],
                preferred_element_type=jnp.float32)
    posc = jnp.dot(pos_ref[...], A[112:, :],
                   preferred_element_type=jnp.float32) + bagg_ref[...]

    half_t = jnp.transpose(half_ref[...])
    io16 = lax.broadcasted_iota(jnp.int32, (16, _S), 0)
    a_item = A[:_DI, :]
    gbs = []
    for b in range(_BB):
        hc = half_t[:, b:b + 1]
        gpair = g[b * _S:(b + 1) * _S]
        gbs.append(jnp.where(hc > 0.5, gpair[:, _DI:], gpair[:, :_DI]))
    gb = jnp.concatenate(gbs, axis=0)
    acc = jnp.dot(gb, a_item, preferred_element_type=jnp.float32)
    for b in range(_BB):
        prow = part_ref[b:b + 1, :]
        crow = corr_ref[b:b + 1, :]
        erow = elap_ref[b:b + 1, :]
        ot = ((io16 == prow).astype(jnp.float32)
              + (io16 == crow + 11).astype(jnp.float32)
              + jnp.where(io16 == 14, erow, 0.0))
        small = lax.dot_general(ot, m, (((0,), (0,)), ((), ())),
                                preferred_element_type=jnp.float32)
        out_ref[b] = acc[b * _S:(b + 1) * _S] + small + posc


def _tc_body_ignore_prev(prev_ref, *rest):
    _tc_body(*rest)


_SMALL_SPECS = [
    pl.BlockSpec((16, 48), lambda i: (0, 0)),
    pl.BlockSpec((_S, 16), lambda i: (0, 0)),
    pl.BlockSpec((_DM, _DM), lambda i: (0, 0)),
    pl.BlockSpec((1, _DM), lambda i: (0, 0)),
]


@jax.jit
def _tc_combine(g2a, g2b, parts, corrs, elaps, halves,
                msrc, positional, wagg, bagg2d):
    blk = [
        pl.BlockSpec((_TROWS, _DM), lambda i: (i, 0)),
        pl.BlockSpec((_BB, _S), lambda i: (i, 0)),
        pl.BlockSpec((_BB, _S), lambda i: (i, 0)),
        pl.BlockSpec((_BB, _S), lambda i: (i, 0)),
        pl.BlockSpec((_BB, _S), lambda i: (i, 0)),
    ]
    out1 = pl.pallas_call(
        _tc_body,
        grid=(_NTILE,),
        in_specs=blk + _SMALL_SPECS,
        out_specs=pl.BlockSpec((_BB, _S, _DM), lambda i: (i, 0, 0)),
        out_shape=jax.ShapeDtypeStruct((_B, _S, _DM), jnp.float32),
    )(g2a, parts[:_BH], corrs[:_BH], elaps[:_BH], halves[:_BH],
      msrc, positional, wagg, bagg2d)
    out = pl.pallas_call(
        _tc_body_ignore_prev,
        grid=(_NTILE,),
        in_specs=[pl.BlockSpec(memory_space=pl.MemorySpace.ANY)] + blk + _SMALL_SPECS,
        out_specs=pl.BlockSpec((_BB, _S, _DM), lambda i: (i + _NTILE, 0, 0)),
        out_shape=jax.ShapeDtypeStruct((_B, _S, _DM), jnp.float32),
        input_output_aliases={0: 0},
    )(out1, g2b, parts[_BH:], corrs[_BH:], elaps[_BH:], halves[_BH:],
      msrc, positional, wagg, bagg2d)
    return out


def kernel(item_id, part_id, is_correct, elapsed_time_norm,
           W_item, W_part, W_correct, W_elapsed, positional, W_agg, b_agg):
    p_table = _tc_pack(W_item.T)

    r = item_id.reshape(_T).astype(jnp.int32)
    p_idx = jnp.where(r < _H, r, r - _H).astype(jnp.int32)
    half = (r >= _H).astype(jnp.float32).reshape(_B, _S)

    g2a = _sc_gather(p_table, p_idx[:_TH])
    g2b = _sc_gather(p_table, p_idx[_TH:])

    msrc = jnp.zeros((16, 48), jnp.float32)
    msrc = msrc.at[0:11, 0:16].set(W_part)
    msrc = msrc.at[11:14, 16:32].set(W_correct)
    msrc = msrc.at[14:15, 32:48].set(W_elapsed)

    elap = elapsed_time_norm.reshape(_B, _S)
    out = _tc_combine(g2a, g2b, part_id.astype(jnp.int32),
                      is_correct.astype(jnp.int32), elap, half,
                      msrc, positional, W_agg, b_agg.reshape(1, _DM))
    return out


# speedup vs baseline: 1.0176x; 1.0176x over previous
import functools

import jax
import jax.numpy as jnp
from jax import lax
from jax.experimental import pallas as pl
from jax.experimental.pallas import tpu as pltpu
from jax.experimental.pallas import tpu_sc as plsc

_B, _S = 1024, 200
_T = _B * _S
_DI = 64
_DM = 128
_V = 1000001

_H = 501760
_PACK_IB = 17920
_NPACK = _H // _PACK_IB

_NW = 32
_NPH = 4
_IDX_COLS = 64
_TH = _T // _NPH
_TOK_PER_W = _TH // _NW
_CHUNK_TOK = 320
_NCHUNK = _TOK_PER_W // _CHUNK_TOK
_GPC = _CHUNK_TOK // _IDX_COLS


def _pack_body(w1_ref, w2_ref, out_ref):
    xs = jnp.concatenate([w1_ref[...], w2_ref[...]], axis=0)
    eye = (lax.broadcasted_iota(jnp.int32, (_DM, _DM), 0)
           == lax.broadcasted_iota(jnp.int32, (_DM, _DM), 1)).astype(jnp.float32)
    out_ref[...] = lax.dot_general(
        xs, eye, (((0,), (0,)), ((), ())),
        preferred_element_type=jnp.float32)


@jax.jit
def _tc_pack(wt):
    return pl.pallas_call(
        _pack_body,
        grid=(_NPACK,),
        in_specs=[
            pl.BlockSpec((_DI, _PACK_IB), lambda i: (0, i)),
            pl.BlockSpec((_DI, _PACK_IB),
                         lambda i: (0, jnp.minimum(_NPACK + i, 2 * _NPACK - 1))),
        ],
        out_specs=pl.BlockSpec((_PACK_IB, _DM), lambda i: (i, 0)),
        out_shape=jax.ShapeDtypeStruct((_H, _DM), jnp.float32),
    )(wt, wt)


def _sc_gather_body(table_hbm, idx_hbm, out_hbm,
                    i0, i1, i2, i3, i4, rows_v, isem, sem):
    nc = 2
    wid = lax.axis_index("s") * nc + lax.axis_index("c")
    tok0 = wid * _TOK_PER_W
    idx_bufs = (i0, i1, i2, i3, i4)
    for g in range(_NCHUNK):
        base = tok0 + g * _CHUNK_TOK
        ic = []
        for j in range(_GPC):
            ic.append(
                pltpu.async_copy(
                    idx_hbm.at[pl.ds(base + j * _IDX_COLS, _IDX_COLS)],
                    idx_bufs[j], isem,
                )
            )
        for c in ic:
            c.wait()
        copies = []
        for j in range(_GPC):
            copies.append(
                pltpu.async_copy(
                    table_hbm.at[idx_bufs[j]],
                    rows_v.at[pl.ds(j * _IDX_COLS, _IDX_COLS)],
                    sem,
                )
            )
        for c in copies:
            c.wait()
        pltpu.sync_copy(rows_v, out_hbm.at[pl.ds(base, _CHUNK_TOK)])


@jax.jit
def _sc_gather(table, idx1d):
    mesh = plsc.VectorSubcoreMesh(core_axis_name="c", subcore_axis_name="s")
    f = functools.partial(
        pl.kernel,
        mesh=mesh,
        out_type=jax.ShapeDtypeStruct((_TH, _DM), jnp.float32),
        scratch_types=[
            pltpu.VMEM((_IDX_COLS,), jnp.int32),
            pltpu.VMEM((_IDX_COLS,), jnp.int32),
            pltpu.VMEM((_IDX_COLS,), jnp.int32),
            pltpu.VMEM((_IDX_COLS,), jnp.int32),
            pltpu.VMEM((_IDX_COLS,), jnp.int32),
            pltpu.VMEM((_CHUNK_TOK, _DM), jnp.float32),
            pltpu.SemaphoreType.DMA,
            pltpu.SemaphoreType.DMA,
        ],
    )(_sc_gather_body)
    return f(table, idx1d)


_BB = 16
_TROWS = _BB * _S
_BH = _B // _NPH
_NTILE = _BH // _BB


def _tc_body(g_ref, part_ref, corr_ref, elap_ref, half_ref, msrc_ref, pos_ref,
             wagg_ref, bagg_ref, out_ref):
    A = wagg_ref[...]
    g = g_ref[...]

    m = jnp.dot(msrc_ref[...], A[_DI:112, :],
                preferred_element_type=jnp.float32)
    posc = jnp.dot(pos_ref[...], A[112:, :],
                   preferred_element_type=jnp.float32) + bagg_ref[...]

    half_t = jnp.transpose(half_ref[...])
    io16 = lax.broadcasted_iota(jnp.int32, (16, _S), 0)
    a_item = A[:_DI, :]
    gbs = []
    for b in range(_BB):
        hc = half_t[:, b:b + 1]
        gpair = g[b * _S:(b + 1) * _S]
        gbs.append(jnp.where(hc > 0.5, gpair[:, _DI:], gpair[:, :_DI]))
    gb = jnp.concatenate(gbs, axis=0)
    acc = jnp.dot(gb, a_item, preferred_element_type=jnp.float32)
    for b in range(_BB):
        prow = part_ref[b:b + 1, :]
        crow = corr_ref[b:b + 1, :]
        erow = elap_ref[b:b + 1, :]
        ot = ((io16 == prow).astype(jnp.float32)
              + (io16 == crow + 11).astype(jnp.float32)
              + jnp.where(io16 == 14, erow, 0.0))
        small = lax.dot_general(ot, m, (((0,), (0,)), ((), ())),
                                preferred_element_type=jnp.float32)
        out_ref[b] = acc[b * _S:(b + 1) * _S] + small + posc


def _tc_body_ignore_prev(prev_ref, *rest):
    _tc_body(*rest)


_SMALL_SPECS = [
    pl.BlockSpec((16, 48), lambda i: (0, 0)),
    pl.BlockSpec((_S, 16), lambda i: (0, 0)),
    pl.BlockSpec((_DM, _DM), lambda i: (0, 0)),
    pl.BlockSpec((1, _DM), lambda i: (0, 0)),
]


def _phase_out_spec(ph):
    return pl.BlockSpec((_BB, _S, _DM), lambda i: (i + ph * _NTILE, 0, 0))


@jax.jit
def _tc_combine(g2s, parts, corrs, elaps, halves,
                msrc, positional, wagg, bagg2d):
    blk = [
        pl.BlockSpec((_TROWS, _DM), lambda i: (i, 0)),
        pl.BlockSpec((_BB, _S), lambda i: (i, 0)),
        pl.BlockSpec((_BB, _S), lambda i: (i, 0)),
        pl.BlockSpec((_BB, _S), lambda i: (i, 0)),
        pl.BlockSpec((_BB, _S), lambda i: (i, 0)),
    ]
    out = None
    for ph in range(_NPH):
        lo, hi = ph * _BH, (ph + 1) * _BH
        args = (g2s[ph], parts[lo:hi], corrs[lo:hi], elaps[lo:hi],
                halves[lo:hi], msrc, positional, wagg, bagg2d)
        if ph == 0:
            out = pl.pallas_call(
                _tc_body,
                grid=(_NTILE,),
                in_specs=blk + _SMALL_SPECS,
                out_specs=_phase_out_spec(0),
                out_shape=jax.ShapeDtypeStruct((_B, _S, _DM), jnp.float32),
            )(*args)
        else:
            out = pl.pallas_call(
                _tc_body_ignore_prev,
                grid=(_NTILE,),
                in_specs=([pl.BlockSpec(memory_space=pl.MemorySpace.ANY)]
                          + blk + _SMALL_SPECS),
                out_specs=_phase_out_spec(ph),
                out_shape=jax.ShapeDtypeStruct((_B, _S, _DM), jnp.float32),
                input_output_aliases={0: 0},
            )(out, *args)
    return out


def kernel(item_id, part_id, is_correct, elapsed_time_norm,
           W_item, W_part, W_correct, W_elapsed, positional, W_agg, b_agg):
    p_table = _tc_pack(W_item.T)

    r = item_id.reshape(_T).astype(jnp.int32)
    p_idx = jnp.where(r < _H, r, r - _H).astype(jnp.int32)
    half = (r >= _H).astype(jnp.float32).reshape(_B, _S)

    g2s = [_sc_gather(p_table, p_idx[ph * _TH:(ph + 1) * _TH])
           for ph in range(_NPH)]

    msrc = jnp.zeros((16, 48), jnp.float32)
    msrc = msrc.at[0:11, 0:16].set(W_part)
    msrc = msrc.at[11:14, 16:32].set(W_correct)
    msrc = msrc.at[14:15, 32:48].set(W_elapsed)

    elap = elapsed_time_norm.reshape(_B, _S)
    out = _tc_combine(g2s, part_id.astype(jnp.int32),
                      is_correct.astype(jnp.int32), elap, half,
                      msrc, positional, W_agg, b_agg.reshape(1, _DM))
    return out


# speedup vs baseline: 1.0423x; 1.0243x over previous
import functools

import jax
import jax.numpy as jnp
from jax import lax
from jax.experimental import pallas as pl
from jax.experimental.pallas import tpu as pltpu
from jax.experimental.pallas import tpu_sc as plsc

_B, _S = 1024, 200
_T = _B * _S
_DI = 64
_DM = 128
_V = 1000001

_H = 501760
_PACK_IB = 17920
_NPACK = _H // _PACK_IB

_NW = 32
_NPH = 4
_IDX_COLS = 64
_TH = _T // _NPH
_TOK_PER_W = _TH // _NW
_CHUNK_TOK = 320
_NCHUNK = _TOK_PER_W // _CHUNK_TOK
_GPC = _CHUNK_TOK // _IDX_COLS


def _pack_body(w1_ref, w2_ref, out_ref):
    xs = jnp.concatenate([w1_ref[...], w2_ref[...]], axis=0)
    eye = (lax.broadcasted_iota(jnp.int32, (_DM, _DM), 0)
           == lax.broadcasted_iota(jnp.int32, (_DM, _DM), 1)).astype(jnp.float32)
    out_ref[...] = lax.dot_general(
        xs, eye, (((0,), (0,)), ((), ())),
        preferred_element_type=jnp.float32)


@jax.jit
def _tc_pack(wt):
    return pl.pallas_call(
        _pack_body,
        grid=(_NPACK,),
        in_specs=[
            pl.BlockSpec((_DI, _PACK_IB), lambda i: (0, i)),
            pl.BlockSpec((_DI, _PACK_IB),
                         lambda i: (0, jnp.minimum(_NPACK + i, 2 * _NPACK - 1))),
        ],
        out_specs=pl.BlockSpec((_PACK_IB, _DM), lambda i: (i, 0)),
        out_shape=jax.ShapeDtypeStruct((_H, _DM), jnp.float32),
    )(wt, wt)


def _sc_gather_body(table_hbm, idx_hbm, out_hbm,
                    i0, i1, i2, i3, i4, rows_v, isem, sem):
    nc = 2
    wid = lax.axis_index("s") * nc + lax.axis_index("c")
    tok0 = wid * _TOK_PER_W
    idx_bufs = (i0, i1, i2, i3, i4)
    for g in range(_NCHUNK):
        base = tok0 + g * _CHUNK_TOK
        ic = []
        for j in range(_GPC):
            ic.append(
                pltpu.async_copy(
                    idx_hbm.at[pl.ds(base + j * _IDX_COLS, _IDX_COLS)],
                    idx_bufs[j], isem,
                )
            )
        for c in ic:
            c.wait()
        copies = []
        for j in range(_GPC):
            copies.append(
                pltpu.async_copy(
                    table_hbm.at[idx_bufs[j]],
                    rows_v.at[pl.ds(j * _IDX_COLS, _IDX_COLS)],
                    sem,
                )
            )
        for c in copies:
            c.wait()
        pltpu.sync_copy(rows_v, out_hbm.at[pl.ds(base, _CHUNK_TOK)])


@jax.jit
def _sc_gather(table, idx1d):
    mesh = plsc.VectorSubcoreMesh(core_axis_name="c", subcore_axis_name="s")
    f = functools.partial(
        pl.kernel,
        mesh=mesh,
        out_type=jax.ShapeDtypeStruct((_TH, _DM), jnp.float32),
        scratch_types=[
            pltpu.VMEM((_IDX_COLS,), jnp.int32),
            pltpu.VMEM((_IDX_COLS,), jnp.int32),
            pltpu.VMEM((_IDX_COLS,), jnp.int32),
            pltpu.VMEM((_IDX_COLS,), jnp.int32),
            pltpu.VMEM((_IDX_COLS,), jnp.int32),
            pltpu.VMEM((_CHUNK_TOK, _DM), jnp.float32),
            pltpu.SemaphoreType.DMA,
            pltpu.SemaphoreType.DMA,
        ],
    )(_sc_gather_body)
    return f(table, idx1d)


_BB = 32
_TROWS = _BB * _S
_BH = _B // _NPH
_NTILE = _BH // _BB


def _tc_body(g_ref, part_ref, corr_ref, elap_ref, half_ref, msrc_ref, pos_ref,
             wagg_ref, bagg_ref, out_ref):
    A = wagg_ref[...]
    g = g_ref[...]

    m = jnp.dot(msrc_ref[...], A[_DI:112, :],
                preferred_element_type=jnp.float32)
    posc = jnp.dot(pos_ref[...], A[112:, :],
                   preferred_element_type=jnp.float32) + bagg_ref[...]

    half_t = jnp.transpose(half_ref[...])
    io16 = lax.broadcasted_iota(jnp.int32, (16, _S), 0)
    a_item = A[:_DI, :]
    gbs = []
    for b in range(_BB):
        hc = half_t[:, b:b + 1]
        gpair = g[b * _S:(b + 1) * _S]
        gbs.append(jnp.where(hc > 0.5, gpair[:, _DI:], gpair[:, :_DI]))
    gb = jnp.concatenate(gbs, axis=0)
    acc = jnp.dot(gb, a_item, preferred_element_type=jnp.float32)
    for b in range(_BB):
        prow = part_ref[b:b + 1, :]
        crow = corr_ref[b:b + 1, :]
        erow = elap_ref[b:b + 1, :]
        ot = ((io16 == prow).astype(jnp.float32)
              + (io16 == crow + 11).astype(jnp.float32)
              + jnp.where(io16 == 14, erow, 0.0))
        small = lax.dot_general(ot, m, (((0,), (0,)), ((), ())),
                                preferred_element_type=jnp.float32)
        out_ref[b] = acc[b * _S:(b + 1) * _S] + small + posc


def _tc_body_ignore_prev(prev_ref, *rest):
    _tc_body(*rest)


_SMALL_SPECS = [
    pl.BlockSpec((16, 48), lambda i: (0, 0)),
    pl.BlockSpec((_S, 16), lambda i: (0, 0)),
    pl.BlockSpec((_DM, _DM), lambda i: (0, 0)),
    pl.BlockSpec((1, _DM), lambda i: (0, 0)),
]


def _phase_out_spec(ph):
    return pl.BlockSpec((_BB, _S, _DM), lambda i: (i + ph * _NTILE, 0, 0))


@jax.jit
def _tc_combine(g2s, parts, corrs, elaps, halves,
                msrc, positional, wagg, bagg2d):
    blk = [
        pl.BlockSpec((_TROWS, _DM), lambda i: (i, 0)),
        pl.BlockSpec((_BB, _S), lambda i: (i, 0)),
        pl.BlockSpec((_BB, _S), lambda i: (i, 0)),
        pl.BlockSpec((_BB, _S), lambda i: (i, 0)),
        pl.BlockSpec((_BB, _S), lambda i: (i, 0)),
    ]
    out = None
    for ph in range(_NPH):
        lo, hi = ph * _BH, (ph + 1) * _BH
        args = (g2s[ph], parts[lo:hi], corrs[lo:hi], elaps[lo:hi],
                halves[lo:hi], msrc, positional, wagg, bagg2d)
        if ph == 0:
            out = pl.pallas_call(
                _tc_body,
                grid=(_NTILE,),
                in_specs=blk + _SMALL_SPECS,
                out_specs=_phase_out_spec(0),
                out_shape=jax.ShapeDtypeStruct((_B, _S, _DM), jnp.float32),
            )(*args)
        else:
            out = pl.pallas_call(
                _tc_body_ignore_prev,
                grid=(_NTILE,),
                in_specs=([pl.BlockSpec(memory_space=pl.MemorySpace.ANY)]
                          + blk + _SMALL_SPECS),
                out_specs=_phase_out_spec(ph),
                out_shape=jax.ShapeDtypeStruct((_B, _S, _DM), jnp.float32),
                input_output_aliases={0: 0},
            )(out, *args)
    return out


def kernel(item_id, part_id, is_correct, elapsed_time_norm,
           W_item, W_part, W_correct, W_elapsed, positional, W_agg, b_agg):
    p_table = _tc_pack(W_item.T)

    r = item_id.reshape(_T).astype(jnp.int32)
    p_idx = jnp.where(r < _H, r, r - _H).astype(jnp.int32)
    half = (r >= _H).astype(jnp.float32).reshape(_B, _S)

    g2s = [_sc_gather(p_table, p_idx[ph * _TH:(ph + 1) * _TH])
           for ph in range(_NPH)]

    msrc = jnp.zeros((16, 48), jnp.float32)
    msrc = msrc.at[0:11, 0:16].set(W_part)
    msrc = msrc.at[11:14, 16:32].set(W_correct)
    msrc = msrc.at[14:15, 32:48].set(W_elapsed)

    elap = elapsed_time_norm.reshape(_B, _S)
    out = _tc_combine(g2s, part_id.astype(jnp.int32),
                      is_correct.astype(jnp.int32), elap, half,
                      msrc, positional, W_agg, b_agg.reshape(1, _DM))
    return out
